# baseline (speedup 1.0000x reference)
import functools

import jax
import jax.numpy as jnp
from jax import lax
from jax.experimental import pallas as pl
from jax.experimental.pallas import tpu as pltpu
from jax.experimental.pallas import tpu_sc as plsc

NUM_USERS = 100000
NUM_ITEMS = 100000
EMBED_DIM = 64
BATCH = 16384

NUM_CORES = 2
NUM_SUBCORES = 16
NW = NUM_CORES * NUM_SUBCORES
BPW = BATCH // NW
CHUNK = 128
NCHUNK = BPW // CHUNK
LANES = 16
NGROUP = BPW // LANES


def _dot_body(users_hbm, items_hbm, ut_hbm, it_hbm, out_hbm,
              uidx, iidx, urows, irows, tbuf, outv, sem):
    wid = lax.axis_index("s") * NUM_CORES + lax.axis_index("c")

    pltpu.sync_copy(users_hbm.at[wid], uidx)
    pltpu.sync_copy(items_hbm.at[wid], iidx)

    copies = []
    for ch in range(NCHUNK):
        dst = pl.ds(ch * CHUNK, CHUNK)
        copies.append(pltpu.async_copy(ut_hbm.at[uidx.at[ch]], urows.at[dst], sem))
        copies.append(pltpu.async_copy(it_hbm.at[iidx.at[ch]], irows.at[dst], sem))
    for cp in copies:
        cp.wait()

    col = lax.iota(jnp.int32, LANES) * LANES

    def group(g, carry):
        for b in range(LANES):
            row = g * LANES + b
            p = jnp.zeros((LANES,), jnp.float32)
            for k in range(EMBED_DIM // LANES):
                u = urows[row, pl.ds(k * LANES, LANES)]
                v = irows[row, pl.ds(k * LANES, LANES)]
                p = p + u * v
            plsc.store_scatter(tbuf, [col + b], p)
        acc = jnp.zeros((LANES,), jnp.float32)
        for r in range(LANES):
            acc = acc + tbuf[pl.ds(r * LANES, LANES)]
        outv[pl.ds(g * LANES, LANES)] = acc
        return carry

    lax.fori_loop(0, NGROUP, group, 0)

    pltpu.sync_copy(outv, out_hbm.at[wid])


@jax.jit
def kernel(users, items, user_table, item_table):
    mesh = plsc.VectorSubcoreMesh(core_axis_name="c", subcore_axis_name="s",
                                  num_cores=NUM_CORES, num_subcores=NUM_SUBCORES)
    run = functools.partial(
        pl.kernel,
        out_type=jax.ShapeDtypeStruct((NW, BPW), jnp.float32),
        mesh=mesh,
        scratch_types=[
            pltpu.VMEM((NCHUNK, CHUNK), jnp.int32),
            pltpu.VMEM((NCHUNK, CHUNK), jnp.int32),
            pltpu.VMEM((BPW, EMBED_DIM), jnp.float32),
            pltpu.VMEM((BPW, EMBED_DIM), jnp.float32),
            pltpu.VMEM((LANES * LANES,), jnp.float32),
            pltpu.VMEM((BPW,), jnp.float32),
            pltpu.SemaphoreType.DMA,
        ],
        compiler_params=pltpu.CompilerParams(needs_layout_passes=False,
                                             use_tc_tiling_on_sc=False),
    )(_dot_body)
    out = run(users.reshape(NW, NCHUNK, CHUNK).astype(jnp.int32),
              items.reshape(NW, NCHUNK, CHUNK).astype(jnp.int32),
              user_table, item_table)
    return out.reshape(BATCH)
